# Initial kernel scaffold; baseline (speedup 1.0000x reference)
import jax
import jax.numpy as jnp
from jax.experimental import pallas as pl


def kernel(x, edge_index, W1, b1, W2, b2):
    raise NotImplementedError("write your pallas kernel here")


# speedup vs baseline: 2.8797x; 2.8797x over previous
import jax
import jax.numpy as jnp
from jax import lax
from jax.experimental import pallas as pl
from jax.experimental.pallas import tpu as pltpu
from jax.experimental.pallas import tpu_sc as plsc

N_NODES = 10000
N_EDGES = 160000
FEATS = 256
FH = 128

NPAD = 10240
DUMMY = N_NODES
NS = 16
K = 128
G = 8
NG = 10
CPT = G * NG
EPAD = NS * CPT * K
ROWS_PT = NPAD // NS


def _sc_aggregate_body(h3_hbm, src_hbm, dst_hbm, agg3_hbm,
                       agg_sh, src_v, dst_v, gbuf, sem):
    c = lax.axis_index("c")
    s = lax.axis_index("s")
    row0 = s * ROWS_PT

    zero16 = jnp.zeros((16,), jnp.float32)

    def fill_gbuf(i, carry):
        for j in range(FH // 16):
            gbuf[i, pl.ds(j * 16, 16)] = zero16
        return carry

    lax.fori_loop(0, K, fill_gbuf, 0)
    for k in range(ROWS_PT // K):
        pltpu.sync_copy(gbuf, agg_sh.at[pl.ds(row0 + k * K, K), :])

    plsc.subcore_barrier()

    def group(g, carry):
        pltpu.sync_copy(src_hbm.at[c, s, pl.ds(g * G, G)], src_v)
        pltpu.sync_copy(dst_hbm.at[s, pl.ds(g * G, G)], dst_v)

        def chunk(i, carry2):
            pltpu.async_copy(h3_hbm.at[src_v.at[i]], gbuf, sem).wait()
            pltpu.sync_copy(gbuf, agg_sh.at[dst_v.at[i]], add=True)
            return carry2

        lax.fori_loop(0, G, chunk, 0)
        return carry

    lax.fori_loop(0, NG, group, 0)

    plsc.subcore_barrier()

    pltpu.sync_copy(agg_sh.at[pl.ds(row0, ROWS_PT), :],
                    agg3_hbm.at[pl.ds(c * NPAD + row0, ROWS_PT), :])


_sc_aggregate = pl.kernel(
    _sc_aggregate_body,
    out_type=jax.ShapeDtypeStruct((2 * NPAD, FH), jnp.float32),
    mesh=plsc.VectorSubcoreMesh(core_axis_name="c", subcore_axis_name="s"),
    scratch_types=[
        pltpu.VMEM_SHARED((NPAD, FH), jnp.float32),
        pltpu.VMEM((G, K), jnp.int32),
        pltpu.VMEM((G, K), jnp.int32),
        pltpu.VMEM((K, FH), jnp.float32),
        pltpu.SemaphoreType.DMA,
    ],
)


def _sc_degree_body(dst_hbm, deg_hbm, deg_sh, dst_v, gbuf):
    c = lax.axis_index("c")
    s = lax.axis_index("s")
    row0 = s * ROWS_PT

    zero16 = jnp.zeros((16,), jnp.float32)
    one16 = jnp.ones((16,), jnp.float32)

    def fill(val):
        def f(i, carry):
            for j in range(FH // 16):
                gbuf[i, pl.ds(j * 16, 16)] = val
            return carry
        return f

    lax.fori_loop(0, K, fill(zero16), 0)
    for k in range(ROWS_PT // K):
        pltpu.sync_copy(gbuf, deg_sh.at[pl.ds(row0 + k * K, K), :])
    lax.fori_loop(0, K, fill(one16), 0)

    plsc.subcore_barrier()

    def group(g, carry):
        pltpu.sync_copy(dst_hbm.at[s, pl.ds(g * G, G)], dst_v)

        def chunk(i, carry2):
            pltpu.sync_copy(gbuf, deg_sh.at[dst_v.at[i]], add=True)
            return carry2

        lax.fori_loop(0, G, chunk, 0)
        return carry

    lax.fori_loop(0, NG, group, 0)

    plsc.subcore_barrier()

    @pl.when(c == 0)
    def _():
        pltpu.sync_copy(deg_sh.at[pl.ds(row0, ROWS_PT), :],
                        deg_hbm.at[pl.ds(row0, ROWS_PT), :])


_sc_degree = pl.kernel(
    _sc_degree_body,
    out_type=jax.ShapeDtypeStruct((NPAD, FH), jnp.float32),
    mesh=plsc.VectorSubcoreMesh(core_axis_name="c", subcore_axis_name="s"),
    scratch_types=[
        pltpu.VMEM_SHARED((NPAD, FH), jnp.float32),
        pltpu.VMEM((G, K), jnp.int32),
        pltpu.VMEM((K, FH), jnp.float32),
    ],
)


TC_ROWS = 1024


def _tc_layer(h3, agg3, deg, wT, b2d):
    def tc_body(h3_ref, agg3_ref, deg_ref, wT_ref, b_ref, o3_ref):
        inv = 1.0 / jnp.maximum(deg_ref[:, 0:1], 1.0)
        out = jnp.dot(h3_ref[0], wT_ref[pl.ds(0, FH), :],
                      preferred_element_type=jnp.float32)
        out += jnp.dot(h3_ref[1], wT_ref[pl.ds(FH, FH), :],
                       preferred_element_type=jnp.float32)
        out += jnp.dot(agg3_ref[0] * inv, wT_ref[pl.ds(2 * FH, FH), :],
                       preferred_element_type=jnp.float32)
        out += jnp.dot(agg3_ref[1] * inv, wT_ref[pl.ds(3 * FH, FH), :],
                       preferred_element_type=jnp.float32)
        out += b_ref[...]
        norm = jnp.sqrt(jnp.sum(out * out, axis=1, keepdims=True))
        out = out / jnp.maximum(norm, 1e-12)
        o3_ref[0] = out[:, :FH]
        o3_ref[1] = out[:, FH:]

    grid = (NPAD // TC_ROWS,)
    return pl.pallas_call(
        tc_body,
        grid=grid,
        in_specs=[
            pl.BlockSpec((2, TC_ROWS, FH), lambda i: (0, i, 0)),
            pl.BlockSpec((2, TC_ROWS, FH), lambda i: (0, i, 0)),
            pl.BlockSpec((TC_ROWS, FH), lambda i: (i, 0)),
            pl.BlockSpec((4 * FH, FEATS), lambda i: (0, 0)),
            pl.BlockSpec((1, FEATS), lambda i: (0, 0)),
        ],
        out_specs=pl.BlockSpec((2, TC_ROWS, FH), lambda i: (0, i, 0)),
        out_shape=jax.ShapeDtypeStruct((2, NPAD, FH), jnp.float32),
    )(h3, agg3, deg, wT, b2d)


def kernel(x, edge_index, W1, b1, W2, b2):
    src = edge_index[0].astype(jnp.int32)
    dst = edge_index[1].astype(jnp.int32)
    src = jnp.concatenate(
        [src, jnp.full((EPAD - N_EDGES,), DUMMY, jnp.int32)]).reshape(NS, CPT, K)
    src = jnp.stack([src, src + NPAD])
    dst = jnp.concatenate(
        [dst, jnp.full((EPAD - N_EDGES,), DUMMY, jnp.int32)]).reshape(NS, CPT, K)

    xp = jnp.pad(x, ((0, NPAD - N_NODES), (0, 0)))
    x2 = jnp.concatenate([xp[:, :FH], xp[:, FH:]], axis=0)

    w1T = W1.T
    w2T = W2.T
    b1r = b1.reshape(1, FEATS)
    b2r = b2.reshape(1, FEATS)

    deg = _sc_degree(dst)
    agg2_1 = _sc_aggregate(x2, src, dst)
    h13 = _tc_layer(x2.reshape(2, NPAD, FH), agg2_1.reshape(2, NPAD, FH),
                    deg, w1T, b1r)
    agg2_2 = _sc_aggregate(h13.reshape(2 * NPAD, FH), src, dst)
    out3 = _tc_layer(h13, agg2_2.reshape(2, NPAD, FH), deg, w2T, b2r)
    return jnp.concatenate([out3[0, :N_NODES], out3[1, :N_NODES]], axis=1)


# speedup vs baseline: 3.2953x; 1.1443x over previous
import jax
import jax.numpy as jnp
from jax import lax
from jax.experimental import pallas as pl
from jax.experimental.pallas import tpu as pltpu
from jax.experimental.pallas import tpu_sc as plsc

N_NODES = 10000
N_EDGES = 160000
FEATS = 256
FH = 128

NPAD = 10240
DUMMY = N_NODES
NS = 16
K = 128
G = 8
NG = 10
CPT = G * NG
EPAD = NS * CPT * K
ROWS_PT = NPAD // NS


def _sc_aggregate_body(h3_hbm, src_hbm, dst_hbm, agg3_hbm,
                       agg_sh, src_v, dst_v, gbuf0, gbuf1, sem0, sem1):
    c = lax.axis_index("c")
    s = lax.axis_index("s")
    row0 = s * ROWS_PT

    zero16 = jnp.zeros((16,), jnp.float32)

    def fill_gbuf(i, carry):
        for j in range(FH // 16):
            gbuf0[i, pl.ds(j * 16, 16)] = zero16
        return carry

    lax.fori_loop(0, K, fill_gbuf, 0)
    for k in range(ROWS_PT // K):
        pltpu.sync_copy(gbuf0, agg_sh.at[pl.ds(row0 + k * K, K), :])

    plsc.subcore_barrier()

    bufs = (gbuf0, gbuf1)
    sems = (sem0, sem1)

    def group(g, carry):
        pltpu.sync_copy(src_hbm.at[c, s, pl.ds(g * G, G)], src_v)
        pltpu.sync_copy(dst_hbm.at[s, pl.ds(g * G, G)], dst_v)

        cps = [None] * G
        cps[0] = pltpu.async_copy(h3_hbm.at[src_v.at[0]], bufs[0], sems[0])
        for i in range(G):
            if i + 1 < G:
                cps[i + 1] = pltpu.async_copy(
                    h3_hbm.at[src_v.at[i + 1]], bufs[(i + 1) % 2],
                    sems[(i + 1) % 2])
            cps[i].wait()
            pltpu.sync_copy(bufs[i % 2], agg_sh.at[dst_v.at[i]], add=True)
        return carry

    lax.fori_loop(0, NG, group, 0)

    plsc.subcore_barrier()

    pltpu.sync_copy(agg_sh.at[pl.ds(row0, ROWS_PT), :],
                    agg3_hbm.at[pl.ds(c * NPAD + row0, ROWS_PT), :])


_sc_aggregate = pl.kernel(
    _sc_aggregate_body,
    out_type=jax.ShapeDtypeStruct((2 * NPAD, FH), jnp.float32),
    mesh=plsc.VectorSubcoreMesh(core_axis_name="c", subcore_axis_name="s"),
    scratch_types=[
        pltpu.VMEM_SHARED((NPAD, FH), jnp.float32),
        pltpu.VMEM((G, K), jnp.int32),
        pltpu.VMEM((G, K), jnp.int32),
        pltpu.VMEM((K, FH), jnp.float32),
        pltpu.VMEM((K, FH), jnp.float32),
        pltpu.SemaphoreType.DMA,
        pltpu.SemaphoreType.DMA,
    ],
)


def _sc_degree_body(dst_hbm, deg_hbm, deg_sh, dst_v, gbuf):
    c = lax.axis_index("c")
    s = lax.axis_index("s")
    row0 = s * ROWS_PT

    zero16 = jnp.zeros((16,), jnp.float32)
    one16 = jnp.ones((16,), jnp.float32)

    def fill(val):
        def f(i, carry):
            for j in range(FH // 16):
                gbuf[i, pl.ds(j * 16, 16)] = val
            return carry
        return f

    lax.fori_loop(0, K, fill(zero16), 0)
    for k in range(ROWS_PT // K):
        pltpu.sync_copy(gbuf, deg_sh.at[pl.ds(row0 + k * K, K), :])
    lax.fori_loop(0, K, fill(one16), 0)

    plsc.subcore_barrier()

    def group(g, carry):
        pltpu.sync_copy(dst_hbm.at[s, pl.ds(g * G, G)], dst_v)

        def chunk(i, carry2):
            pltpu.sync_copy(gbuf, deg_sh.at[dst_v.at[i]], add=True)
            return carry2

        lax.fori_loop(0, G, chunk, 0)
        return carry

    lax.fori_loop(0, NG, group, 0)

    plsc.subcore_barrier()

    @pl.when(c == 0)
    def _():
        pltpu.sync_copy(deg_sh.at[pl.ds(row0, ROWS_PT), :],
                        deg_hbm.at[pl.ds(row0, ROWS_PT), :])


_sc_degree = pl.kernel(
    _sc_degree_body,
    out_type=jax.ShapeDtypeStruct((NPAD, FH), jnp.float32),
    mesh=plsc.VectorSubcoreMesh(core_axis_name="c", subcore_axis_name="s"),
    scratch_types=[
        pltpu.VMEM_SHARED((NPAD, FH), jnp.float32),
        pltpu.VMEM((G, K), jnp.int32),
        pltpu.VMEM((K, FH), jnp.float32),
    ],
)


TC_ROWS = 1024


def _tc_layer(h3, agg3, deg, wT, b2d):
    def tc_body(h3_ref, agg3_ref, deg_ref, wT_ref, b_ref, o3_ref):
        inv = 1.0 / jnp.maximum(deg_ref[:, 0:1], 1.0)
        out = jnp.dot(h3_ref[0], wT_ref[pl.ds(0, FH), :],
                      preferred_element_type=jnp.float32)
        out += jnp.dot(h3_ref[1], wT_ref[pl.ds(FH, FH), :],
                       preferred_element_type=jnp.float32)
        out += jnp.dot(agg3_ref[0] * inv, wT_ref[pl.ds(2 * FH, FH), :],
                       preferred_element_type=jnp.float32)
        out += jnp.dot(agg3_ref[1] * inv, wT_ref[pl.ds(3 * FH, FH), :],
                       preferred_element_type=jnp.float32)
        out += b_ref[...]
        norm = jnp.sqrt(jnp.sum(out * out, axis=1, keepdims=True))
        out = out / jnp.maximum(norm, 1e-12)
        o3_ref[0] = out[:, :FH]
        o3_ref[1] = out[:, FH:]

    grid = (NPAD // TC_ROWS,)
    return pl.pallas_call(
        tc_body,
        grid=grid,
        in_specs=[
            pl.BlockSpec((2, TC_ROWS, FH), lambda i: (0, i, 0)),
            pl.BlockSpec((2, TC_ROWS, FH), lambda i: (0, i, 0)),
            pl.BlockSpec((TC_ROWS, FH), lambda i: (i, 0)),
            pl.BlockSpec((4 * FH, FEATS), lambda i: (0, 0)),
            pl.BlockSpec((1, FEATS), lambda i: (0, 0)),
        ],
        out_specs=pl.BlockSpec((2, TC_ROWS, FH), lambda i: (0, i, 0)),
        out_shape=jax.ShapeDtypeStruct((2, NPAD, FH), jnp.float32),
    )(h3, agg3, deg, wT, b2d)


def kernel(x, edge_index, W1, b1, W2, b2):
    src = edge_index[0].astype(jnp.int32)
    dst = edge_index[1].astype(jnp.int32)
    src = jnp.concatenate(
        [src, jnp.full((EPAD - N_EDGES,), DUMMY, jnp.int32)]).reshape(NS, CPT, K)
    src = jnp.stack([src, src + NPAD])
    dst = jnp.concatenate(
        [dst, jnp.full((EPAD - N_EDGES,), DUMMY, jnp.int32)]).reshape(NS, CPT, K)

    xp = jnp.pad(x, ((0, NPAD - N_NODES), (0, 0)))
    x2 = jnp.concatenate([xp[:, :FH], xp[:, FH:]], axis=0)

    w1T = W1.T
    w2T = W2.T
    b1r = b1.reshape(1, FEATS)
    b2r = b2.reshape(1, FEATS)

    deg = _sc_degree(dst)
    agg2_1 = _sc_aggregate(x2, src, dst)
    h13 = _tc_layer(x2.reshape(2, NPAD, FH), agg2_1.reshape(2, NPAD, FH),
                    deg, w1T, b1r)
    agg2_2 = _sc_aggregate(h13.reshape(2 * NPAD, FH), src, dst)
    out3 = _tc_layer(h13, agg2_2.reshape(2, NPAD, FH), deg, w2T, b2r)
    return jnp.concatenate([out3[0, :N_NODES], out3[1, :N_NODES]], axis=1)
